# baseline (speedup 1.0000x reference)
import functools

import jax
import jax.numpy as jnp
from jax import lax
from jax.experimental import pallas as pl
from jax.experimental.pallas import tpu as pltpu
from jax.experimental.pallas import tpu_sc as plsc

B = 16384
D = 64
L = 16

_info = plsc.get_sparse_core_info()
NC = _info.num_cores
NS = _info.num_subcores
NW = NC * NS
PW = B // NW
CH = 128
NCH = PW // CH
GROUPS = PW // L


def _trans_e_body(h_hbm, r_hbm, t_hbm, ent_hbm, rel_hbm, out_hbm,
                  hi, ri, ti, hrows, rrows, trows, outv, sem):
    wid = lax.axis_index("s") * NC + lax.axis_index("c")

    pltpu.sync_copy(h_hbm.at[wid], hi)
    pltpu.sync_copy(r_hbm.at[wid], ri)
    pltpu.sync_copy(t_hbm.at[wid], ti)

    copies = []
    for j in range(NCH):
        copies.append(pltpu.async_copy(
            ent_hbm.at[hi.at[j]], hrows.at[pl.ds(j * CH, CH)], sem))
        copies.append(pltpu.async_copy(
            rel_hbm.at[ri.at[j]], rrows.at[pl.ds(j * CH, CH)], sem))
        copies.append(pltpu.async_copy(
            ent_hbm.at[ti.at[j]], trows.at[pl.ds(j * CH, CH)], sem))
    for c in copies:
        c.wait()

    lane = lax.iota(jnp.int32, L)

    def group_body(g, _):
        row = lane + g * L
        acc = jnp.zeros((L,), jnp.float32)
        for c in range(D):
            col = jnp.full((L,), c, jnp.int32)
            hv = plsc.load_gather(hrows, [row, col])
            rv = plsc.load_gather(rrows, [row, col])
            tv = plsc.load_gather(trows, [row, col])
            acc = acc + jnp.abs(hv + rv - tv)
        outv[pl.ds(g * L, L)] = acc
        return 0

    lax.fori_loop(0, GROUPS, group_body, 0)

    pltpu.sync_copy(outv, out_hbm.at[pl.ds(wid * PW, PW)])


@jax.jit
def _trans_e(h, r, t, entity_emb, relation_emb):
    mesh = plsc.VectorSubcoreMesh(core_axis_name="c", subcore_axis_name="s")
    run = functools.partial(
        pl.kernel,
        mesh=mesh,
        compiler_params=pltpu.CompilerParams(
            needs_layout_passes=False, use_tc_tiling_on_sc=False),
        out_type=jax.ShapeDtypeStruct((B,), jnp.float32),
        scratch_types=[
            pltpu.VMEM((NCH, CH), jnp.int32),
            pltpu.VMEM((NCH, CH), jnp.int32),
            pltpu.VMEM((NCH, CH), jnp.int32),
            pltpu.VMEM((PW, D), jnp.float32),
            pltpu.VMEM((PW, D), jnp.float32),
            pltpu.VMEM((PW, D), jnp.float32),
            pltpu.VMEM((PW,), jnp.float32),
            pltpu.SemaphoreType.DMA,
        ],
    )(_trans_e_body)
    return run(h, r, t, entity_emb, relation_emb)


def kernel(h, r, t, entity_emb, relation_emb):
    h3 = h.astype(jnp.int32).reshape(NW, NCH, CH)
    r3 = r.astype(jnp.int32).reshape(NW, NCH, CH)
    t3 = t.astype(jnp.int32).reshape(NW, NCH, CH)
    return _trans_e(h3, r3, t3, entity_emb, relation_emb)


# speedup vs baseline: 1.5821x; 1.5821x over previous
import functools

import jax
import jax.numpy as jnp
from jax import lax
from jax.experimental import pallas as pl
from jax.experimental.pallas import tpu as pltpu
from jax.experimental.pallas import tpu_sc as plsc

B = 16384
D = 64
L = 16

_info = plsc.get_sparse_core_info()
NC = _info.num_cores
NS = _info.num_subcores
NW = NC * NS
PW = B // NW
CPR = 128
NCK = PW // CPR
CGRP = CPR // L


def _trans_e_body(h_hbm, r_hbm, t_hbm, ent_hbm, rel_hbm, out_hbm,
                  him, rim, tim,
                  hb, rb, tb, outv, sem):
    wid = lax.axis_index("s") * NC + lax.axis_index("c")

    pltpu.sync_copy(h_hbm.at[wid], him)
    pltpu.sync_copy(r_hbm.at[wid], rim)
    pltpu.sync_copy(t_hbm.at[wid], tim)

    lane = lax.iota(jnp.int32, L)

    def chunk_body(k, _):
        base = k * CPR

        def fire_body(q, _):
            hv16 = him[pl.ds(base + q * L, L)]
            rv16 = rim[pl.ds(base + q * L, L)]
            tv16 = tim[pl.ds(base + q * L, L)]
            for jj in range(L):
                i = q * L + jj
                pltpu.async_copy(ent_hbm.at[hv16[jj]], hb.at[i], sem)
                pltpu.async_copy(rel_hbm.at[rv16[jj]], rb.at[i], sem)
                pltpu.async_copy(ent_hbm.at[tv16[jj]], tb.at[i], sem)
            return 0

        lax.fori_loop(0, CPR // L, fire_body, 0)

        def drain_body(i, _):
            pltpu.make_async_copy(ent_hbm.at[0], hb.at[0], sem).wait()
            pltpu.make_async_copy(rel_hbm.at[0], rb.at[0], sem).wait()
            pltpu.make_async_copy(ent_hbm.at[0], tb.at[0], sem).wait()
            return 0

        lax.fori_loop(0, CPR, drain_body, 0)

        def group_body(g, _):
            row = lane + g * L
            acc = jnp.zeros((L,), jnp.float32)
            for c in range(D):
                col = jnp.full((L,), c, jnp.int32)
                hv = plsc.load_gather(hb, [row, col])
                rv = plsc.load_gather(rb, [row, col])
                tv = plsc.load_gather(tb, [row, col])
                acc = acc + jnp.abs(hv + rv - tv)
            outv[pl.ds(base + g * L, L)] = acc
            return 0

        lax.fori_loop(0, CGRP, group_body, 0)
        return 0

    lax.fori_loop(0, NCK, chunk_body, 0)

    pltpu.sync_copy(outv, out_hbm.at[pl.ds(wid * PW, PW)])


@jax.jit
def _trans_e(h, r, t, entity_emb, relation_emb):
    mesh = plsc.VectorSubcoreMesh(core_axis_name="c", subcore_axis_name="s")
    run = functools.partial(
        pl.kernel,
        mesh=mesh,
        compiler_params=pltpu.CompilerParams(needs_layout_passes=False),
        out_type=jax.ShapeDtypeStruct((B,), jnp.float32),
        scratch_types=[
            pltpu.VMEM((PW,), jnp.int32),
            pltpu.VMEM((PW,), jnp.int32),
            pltpu.VMEM((PW,), jnp.int32),
            pltpu.VMEM((CPR, D), jnp.float32),
            pltpu.VMEM((CPR, D), jnp.float32),
            pltpu.VMEM((CPR, D), jnp.float32),
            pltpu.VMEM((PW,), jnp.float32),
            pltpu.SemaphoreType.DMA,
        ],
    )(_trans_e_body)
    return run(h, r, t, entity_emb, relation_emb)


def kernel(h, r, t, entity_emb, relation_emb):
    h2 = h.astype(jnp.int32).reshape(NW, PW)
    r2 = r.astype(jnp.int32).reshape(NW, PW)
    t2 = t.astype(jnp.int32).reshape(NW, PW)
    return _trans_e(h2, r2, t2, entity_emb, relation_emb)
